# Initial kernel scaffold; baseline (speedup 1.0000x reference)
import jax
import jax.numpy as jnp
from jax.experimental import pallas as pl


def kernel(target_masks, target_class_ids, pred_masks):
    raise NotImplementedError("write your pallas kernel here")


# speedup vs baseline: 2.0878x; 2.0878x over previous
import functools

import jax
import jax.numpy as jnp
from jax import lax
from jax.experimental import pallas as pl
from jax.experimental.pallas import tpu as pltpu
from jax.experimental.pallas import tpu_sc as plsc

_N = 400
_HW = 784
_C = 81
_CH = 7
_CW = 112
_NW = 32
_RPW = 13


def _sc_gather(pred_flat, tci):
    mesh = plsc.VectorSubcoreMesh(core_axis_name="c", subcore_axis_name="s")

    @functools.partial(
        pl.kernel,
        mesh=mesh,
        out_type=jax.ShapeDtypeStruct((_N, _CH, _CW), jnp.float32),
        scratch_types=[
            pltpu.VMEM((_N + 16,), jnp.int32),
            pltpu.VMEM((_CH, _CW), jnp.int32),
            pltpu.VMEM((_CH, _CW), jnp.float32),
            pltpu.SemaphoreType.DMA,
        ],
    )
    def k(pred_hbm, tci_hbm, out_hbm, tci_v, idx_v, yp_v, sem):
        wid = lax.axis_index("s") * 2 + lax.axis_index("c")
        pltpu.sync_copy(tci_hbm, tci_v.at[pl.ds(0, _N)])
        iota = lax.iota(jnp.int32, 16)

        def body(r, carry):
            i = wid + r * _NW

            @pl.when(i < _N)
            def _():
                c_vec = tci_v[pl.ds(i, 16)]
                base = c_vec[0] + i * (_HW * _C)
                for j in range(_CH):
                    for g in range(_CW // 16):
                        kk = j * _CW + g * 16
                        idx_v[j, pl.ds(g * 16, 16)] = base + (kk + iota) * _C
                copies = [
                    pltpu.async_copy(pred_hbm.at[idx_v.at[j]], yp_v.at[j], sem)
                    for j in range(_CH)
                ]
                for cp in copies:
                    cp.wait()
                pltpu.sync_copy(yp_v, out_hbm.at[i])

            return carry

        lax.fori_loop(0, _RPW, body, 0)

    return k(pred_flat, tci)


def _tc_loss(ypred, ytrue, cls2d):
    def body(yp_ref, yt_ref, cls_ref, out_ref):
        eps = jnp.float32(1e-7)
        p = jnp.clip(yp_ref[...], eps, jnp.float32(1.0) - eps)
        y = yt_ref[...]
        bce = -(y * jnp.log(p) + (jnp.float32(1.0) - y) * jnp.log(jnp.float32(1.0) - p))
        vf = (cls_ref[...] > 0).astype(jnp.float32)
        s = jnp.sum(bce * vf)
        cnt = jnp.sum(vf)
        denom = cnt * jnp.float32(_HW)
        out_ref[0, 0] = jnp.where(cnt > 0, s / denom, jnp.float32(0.0))

    return pl.pallas_call(
        body,
        out_shape=jax.ShapeDtypeStruct((1, 1), jnp.float32),
        out_specs=pl.BlockSpec(memory_space=pltpu.SMEM),
    )(ypred, ytrue, cls2d)


def kernel(target_masks, target_class_ids, pred_masks):
    tci = target_class_ids.reshape(-1)
    yp = _sc_gather(pred_masks.reshape(-1), tci)
    loss = _tc_loss(
        yp.reshape(_N, _HW),
        target_masks.reshape(_N, _HW),
        tci.reshape(_N, 1),
    )
    return loss[0, 0]


# speedup vs baseline: 6.5284x; 3.1270x over previous
import jax
import jax.numpy as jnp
from jax.experimental import pallas as pl
from jax.experimental.pallas import tpu as pltpu

_B, _R = 4, 100
_N = _B * _R
_HW = 784
_C = 81
_PPB = 8
_G = _HW // _PPB


def _loss_kernel(cls_ref, pred_ref, tm_ref, out_ref, oh_ref, acc_ref):
    g = pl.program_id(0)

    @pl.when(g == 0)
    def _():
        cls = cls_ref[...]
        cid = jax.lax.broadcasted_iota(jnp.int32, (_C, _B, _R), 0)
        oh_ref[...] = (cid == cls[None, :, :]).astype(jnp.float32)
        acc_ref[0] = jnp.float32(0.0)

    x = pred_ref[...].reshape(_PPB, _C, _B, _R)
    yp = jnp.sum(x * oh_ref[...][None], axis=1)
    eps = jnp.float32(1e-7)
    p = jnp.clip(yp, eps, jnp.float32(1.0) - eps)
    y = tm_ref[...]
    bce = -(y * jnp.log(p) + (jnp.float32(1.0) - y) * jnp.log(jnp.float32(1.0) - p))
    vf = (cls_ref[...] > 0).astype(jnp.float32)
    acc_ref[0] += jnp.sum(bce * vf[None, :, :])

    @pl.when(g == _G - 1)
    def _():
        cnt = jnp.sum((cls_ref[...] > 0).astype(jnp.float32))
        denom = cnt * jnp.float32(_HW)
        out_ref[0, 0] = jnp.where(cnt > 0, acc_ref[0] / denom, jnp.float32(0.0))


def kernel(target_masks, target_class_ids, pred_masks):
    pred_v = jnp.transpose(pred_masks, (2, 3, 4, 0, 1)).reshape(_HW * _C, _B, _R)
    tm_v = jnp.transpose(target_masks, (2, 3, 0, 1)).reshape(_HW, _B, _R)

    loss = pl.pallas_call(
        _loss_kernel,
        grid=(_G,),
        in_specs=[
            pl.BlockSpec((_B, _R), lambda g: (0, 0)),
            pl.BlockSpec((_PPB * _C, _B, _R), lambda g: (g, 0, 0)),
            pl.BlockSpec((_PPB, _B, _R), lambda g: (g, 0, 0)),
        ],
        out_specs=pl.BlockSpec(memory_space=pltpu.SMEM),
        out_shape=jax.ShapeDtypeStruct((1, 1), jnp.float32),
        scratch_shapes=[
            pltpu.VMEM((_C, _B, _R), jnp.float32),
            pltpu.SMEM((1,), jnp.float32),
        ],
    )(target_class_ids, pred_v, tm_v)
    return loss[0, 0]


# speedup vs baseline: 6.9595x; 1.0660x over previous
import jax
import jax.numpy as jnp
from jax.experimental import pallas as pl
from jax.experimental.pallas import tpu as pltpu

_B, _R = 4, 100
_HW = 784
_C = 81
_PPB = 8
_G = _HW // _PPB
_ROWS = _PPB * _C * _B
_CB = _C * _B
_QR = _PPB * _B


def _loss_kernel(cls_ref, pred_ref, tm_ref, out_ref, oh_ref, sel_ref, vm_ref, acc_ref):
    g = pl.program_id(0)

    @pl.when(g == 0)
    def _():
        cls = cls_ref[...]
        cid = jax.lax.broadcasted_iota(jnp.int32, (_C, _B, _R), 0)
        oh1 = (cid == cls[None, :, :]).astype(jnp.float32).reshape(_CB, _R)
        vm1 = (cls > 0).astype(jnp.float32)
        for q in range(_PPB):
            oh_ref[pl.ds(q * _CB, _CB), :] = oh1
            vm_ref[pl.ds(q * _B, _B), :] = vm1
        iq = jax.lax.broadcasted_iota(jnp.int32, (_QR, _ROWS), 0)
        ij = jax.lax.broadcasted_iota(jnp.int32, (_QR, _ROWS), 1)
        sel_ref[...] = ((ij // _CB == iq // _B) & (ij % _B == iq % _B)).astype(
            jnp.float32
        )
        acc_ref[0] = jnp.float32(0.0)

    xm = pred_ref[...] * oh_ref[...]
    yp = jax.lax.dot(sel_ref[...], xm,
                     preferred_element_type=jnp.float32)
    eps = jnp.float32(1e-7)
    p = jnp.clip(yp, eps, jnp.float32(1.0) - eps)
    y = tm_ref[...]
    bce = -(y * jnp.log(p) + (jnp.float32(1.0) - y) * jnp.log(jnp.float32(1.0) - p))
    acc_ref[0] += jnp.sum(bce * vm_ref[...])

    @pl.when(g == _G - 1)
    def _():
        cnt = jnp.sum((cls_ref[...] > 0).astype(jnp.float32))
        denom = cnt * jnp.float32(_HW)
        out_ref[0, 0] = jnp.where(cnt > 0, acc_ref[0] / denom, jnp.float32(0.0))


def kernel(target_masks, target_class_ids, pred_masks):
    pred_v = jnp.transpose(pred_masks, (2, 3, 4, 0, 1)).reshape(_HW * _CB, _R)
    tm_v = jnp.transpose(target_masks, (2, 3, 0, 1)).reshape(_HW * _B, _R)

    loss = pl.pallas_call(
        _loss_kernel,
        grid=(_G,),
        in_specs=[
            pl.BlockSpec((_B, _R), lambda g: (0, 0)),
            pl.BlockSpec((_ROWS, _R), lambda g: (g, 0)),
            pl.BlockSpec((_QR, _R), lambda g: (g, 0)),
        ],
        out_specs=pl.BlockSpec(memory_space=pltpu.SMEM),
        out_shape=jax.ShapeDtypeStruct((1, 1), jnp.float32),
        scratch_shapes=[
            pltpu.VMEM((_ROWS, _R), jnp.float32),
            pltpu.VMEM((_QR, _ROWS), jnp.float32),
            pltpu.VMEM((_QR, _R), jnp.float32),
            pltpu.SMEM((1,), jnp.float32),
        ],
    )(target_class_ids, pred_v, tm_v)
    return loss[0, 0]


# speedup vs baseline: 11.1069x; 1.5959x over previous
import jax
import jax.numpy as jnp
from jax.experimental import pallas as pl
from jax.experimental.pallas import tpu as pltpu

_B, _R = 4, 100
_HW = 784
_C = 81
_PPB = 28
_G = _HW // _PPB
_CB = _C * _B
_ROWS = _PPB * _CB
_QR = _PPB * _B


def _loss_kernel(cls_ref, pred_ref, tm_ref, out_ref, oh_ref, vm_ref, acc_ref):
    g = pl.program_id(0)

    @pl.when(g == 0)
    def _():
        cls = cls_ref[...]
        cid = jax.lax.broadcasted_iota(jnp.int32, (_C, _B, _R), 0)
        oh1 = (cid == cls[None, :, :]).astype(jnp.float32).reshape(_CB, _R)
        vm1 = (cls > 0).astype(jnp.float32)
        for q in range(_PPB):
            oh_ref[pl.ds(q * _CB, _CB), :] = oh1
            vm_ref[pl.ds(q * _B, _B), :] = vm1
        acc_ref[0] = jnp.float32(0.0)

    xm = pred_ref[...] * oh_ref[...]
    yp = jnp.sum(xm.reshape(_PPB, _C, _B, _R), axis=1)
    yp = yp.reshape(_QR, _R)
    eps = jnp.float32(1e-7)
    p = jnp.clip(yp, eps, jnp.float32(1.0) - eps)
    y = tm_ref[...]
    bce = -(y * jnp.log(p) + (jnp.float32(1.0) - y) * jnp.log(jnp.float32(1.0) - p))
    acc_ref[0] += jnp.sum(bce * vm_ref[...])

    @pl.when(g == _G - 1)
    def _():
        cnt = jnp.sum((cls_ref[...] > 0).astype(jnp.float32))
        denom = cnt * jnp.float32(_HW)
        out_ref[0, 0] = jnp.where(cnt > 0, acc_ref[0] / denom, jnp.float32(0.0))


def kernel(target_masks, target_class_ids, pred_masks):
    pred_v = jnp.transpose(pred_masks, (2, 3, 4, 0, 1)).reshape(_HW * _CB, _R)
    tm_v = jnp.transpose(target_masks, (2, 3, 0, 1)).reshape(_HW * _B, _R)

    loss = pl.pallas_call(
        _loss_kernel,
        grid=(_G,),
        in_specs=[
            pl.BlockSpec((_B, _R), lambda g: (0, 0)),
            pl.BlockSpec((_ROWS, _R), lambda g: (g, 0)),
            pl.BlockSpec((_QR, _R), lambda g: (g, 0)),
        ],
        out_specs=pl.BlockSpec(memory_space=pltpu.SMEM),
        out_shape=jax.ShapeDtypeStruct((1, 1), jnp.float32),
        scratch_shapes=[
            pltpu.VMEM((_ROWS, _R), jnp.float32),
            pltpu.VMEM((_QR, _R), jnp.float32),
            pltpu.SMEM((1,), jnp.float32),
        ],
    )(target_class_ids, pred_v, tm_v)
    return loss[0, 0]


# speedup vs baseline: 12.3056x; 1.1079x over previous
import jax
import jax.numpy as jnp
from jax.experimental import pallas as pl
from jax.experimental.pallas import tpu as pltpu

_B, _R = 4, 100
_HW = 784
_C = 81
_PPB = 56
_G = _HW // _PPB
_CB = _C * _B
_ROWS = _PPB * _CB
_QR = _PPB * _B


def _loss_kernel(cls_ref, pred_ref, tm_ref, out_ref, oh_ref, vm_ref, acc_ref):
    g = pl.program_id(0)

    @pl.when(g == 0)
    def _():
        cls = cls_ref[...]
        cid = jax.lax.broadcasted_iota(jnp.int32, (_C, _B, _R), 0)
        oh1 = (cid == cls[None, :, :]).astype(jnp.float32).reshape(_CB, _R)
        vm1 = (cls > 0).astype(jnp.float32)
        for q in range(_PPB):
            oh_ref[pl.ds(q * _CB, _CB), :] = oh1
            vm_ref[pl.ds(q * _B, _B), :] = vm1
        acc_ref[0] = jnp.float32(0.0)

    xm = pred_ref[...] * oh_ref[...]
    yp = jnp.sum(xm.reshape(_PPB, _C, _B, _R), axis=1)
    yp = yp.reshape(_QR, _R)
    eps = jnp.float32(1e-7)
    p = jnp.clip(yp, eps, jnp.float32(1.0) - eps)
    y = tm_ref[...]
    bce = -(y * jnp.log(p) + (jnp.float32(1.0) - y) * jnp.log(jnp.float32(1.0) - p))
    acc_ref[0] += jnp.sum(bce * vm_ref[...])

    @pl.when(g == _G - 1)
    def _():
        cnt = jnp.sum((cls_ref[...] > 0).astype(jnp.float32))
        denom = cnt * jnp.float32(_HW)
        out_ref[0, 0] = jnp.where(cnt > 0, acc_ref[0] / denom, jnp.float32(0.0))


def kernel(target_masks, target_class_ids, pred_masks):
    pred_v = jnp.transpose(pred_masks, (2, 3, 4, 0, 1)).reshape(_HW * _CB, _R)
    tm_v = jnp.transpose(target_masks, (2, 3, 0, 1)).reshape(_HW * _B, _R)

    loss = pl.pallas_call(
        _loss_kernel,
        grid=(_G,),
        in_specs=[
            pl.BlockSpec((_B, _R), lambda g: (0, 0)),
            pl.BlockSpec((_ROWS, _R), lambda g: (g, 0)),
            pl.BlockSpec((_QR, _R), lambda g: (g, 0)),
        ],
        out_specs=pl.BlockSpec(memory_space=pltpu.SMEM),
        out_shape=jax.ShapeDtypeStruct((1, 1), jnp.float32),
        scratch_shapes=[
            pltpu.VMEM((_ROWS, _R), jnp.float32),
            pltpu.VMEM((_QR, _R), jnp.float32),
            pltpu.SMEM((1,), jnp.float32),
        ],
    )(target_class_ids, pred_v, tm_v)
    return loss[0, 0]


# speedup vs baseline: 13.3668x; 1.0862x over previous
import jax
import jax.numpy as jnp
from jax.experimental import pallas as pl
from jax.experimental.pallas import tpu as pltpu

_B, _R = 4, 100
_HW = 784
_C = 81
_PPB = 112
_G = _HW // _PPB
_CB = _C * _B
_ROWS = _PPB * _CB
_QR = _PPB * _B
_PCH = 2
_CHR = _PCH * _CB
_CHQ = _PCH * _B


def _loss_kernel(cls_ref, pred_ref, tm_ref, out_ref, oh_ref, vm_ref, acc_ref):
    g = pl.program_id(0)

    @pl.when(g == 0)
    def _():
        cls = cls_ref[...]
        cid = jax.lax.broadcasted_iota(jnp.int32, (_C, _B, _R), 0)
        oh1 = (cid == cls[None, :, :]).astype(jnp.float32).reshape(_CB, _R)
        vm1 = (cls > 0).astype(jnp.float32)
        for q in range(_PCH):
            oh_ref[pl.ds(q * _CB, _CB), :] = oh1
            vm_ref[pl.ds(q * _B, _B), :] = vm1
        acc_ref[...] = jnp.zeros((_CHQ, _R), jnp.float32)

    eps = jnp.float32(1e-7)
    one = jnp.float32(1.0)
    oh = oh_ref[...]
    vm = vm_ref[...]
    for q in range(_PPB // _PCH):
        xm = pred_ref[pl.ds(q * _CHR, _CHR), :] * oh
        yp = jnp.sum(xm.reshape(_PCH, _C, _B, _R), axis=1)
        p = jnp.clip(yp.reshape(_CHQ, _R), eps, one - eps)
        y = tm_ref[pl.ds(q * _CHQ, _CHQ), :]
        bce = -(y * jnp.log(p) + (one - y) * jnp.log(one - p))
        acc_ref[...] += bce * vm

    @pl.when(g == _G - 1)
    def _():
        cnt = jnp.sum((cls_ref[...] > 0).astype(jnp.float32))
        denom = cnt * jnp.float32(_HW)
        total = jnp.sum(acc_ref[...])
        out_ref[0, 0] = jnp.where(cnt > 0, total / denom, jnp.float32(0.0))


def kernel(target_masks, target_class_ids, pred_masks):
    pred_v = jnp.transpose(pred_masks, (2, 3, 4, 0, 1)).reshape(_HW * _CB, _R)
    tm_v = jnp.transpose(target_masks, (2, 3, 0, 1)).reshape(_HW * _B, _R)

    loss = pl.pallas_call(
        _loss_kernel,
        grid=(_G,),
        in_specs=[
            pl.BlockSpec((_B, _R), lambda g: (0, 0)),
            pl.BlockSpec((_ROWS, _R), lambda g: (g, 0)),
            pl.BlockSpec((_QR, _R), lambda g: (g, 0)),
        ],
        out_specs=pl.BlockSpec(memory_space=pltpu.SMEM),
        out_shape=jax.ShapeDtypeStruct((1, 1), jnp.float32),
        scratch_shapes=[
            pltpu.VMEM((_CHR, _R), jnp.float32),
            pltpu.VMEM((_CHQ, _R), jnp.float32),
            pltpu.VMEM((_CHQ, _R), jnp.float32),
        ],
    )(target_class_ids, pred_v, tm_v)
    return loss[0, 0]
